# Initial kernel scaffold; baseline (speedup 1.0000x reference)
import jax
import jax.numpy as jnp
from jax.experimental import pallas as pl


def kernel(loc_preds, loc_targets, cls_preds, cls_targets):
    raise NotImplementedError("write your pallas kernel here")


# speedup vs baseline: 1.6404x; 1.6404x over previous
import functools

import jax
import jax.numpy as jnp
from jax.experimental import pallas as pl
from jax.experimental.pallas import tpu as pltpu

_N = 32
_A = 8732
_C = 81


def _phase1_body(cls_ref, tgt_ref, lse_ref, tl_ref):
    x = cls_ref[0]
    t = tgt_ref[0]
    m = jnp.max(x, axis=1, keepdims=True)
    e = jnp.exp(x - m)
    s = jnp.sum(e, axis=1, keepdims=True)
    lse_ref[0] = m + jnp.log(s)
    cio = jax.lax.broadcasted_iota(jnp.int32, (_A, _C), 1)
    tl = jnp.sum(jnp.where(cio == t, x, 0.0), axis=1, keepdims=True)
    tl_ref[0] = tl


def _phase2_body(lse_ref, tl_ref, ct_ref, lp_ref, lt_ref, out_ref):
    lse = lse_ref[...]
    tl = tl_ref[...]
    ct = ct_ref[...]
    pos = ct > 0
    posf = pos.astype(jnp.float32)

    cl = jnp.maximum(lse - tl, 0.0)
    v = jnp.where(pos, 0.0, cl)

    np_i = jnp.sum(pos.astype(jnp.int32), axis=1, keepdims=True)
    k = jnp.minimum(3 * np_i, _A)
    pcl = jnp.sum(cl * posf, axis=1, keepdims=True)
    sumv = jnp.sum(v, axis=1, keepdims=True)

    def bit_step(i, p):
        cand = p | (1 << (30 - i))
        tval = jax.lax.bitcast_convert_type(cand, jnp.float32)
        cnt = jnp.sum((v >= tval).astype(jnp.int32), axis=1, keepdims=True)
        return jnp.where(cnt >= k, cand, p)

    p = jax.lax.fori_loop(0, 31, bit_step, jnp.zeros((_N, 1), jnp.int32))
    tval = jax.lax.bitcast_convert_type(p, jnp.float32)
    gt = v > tval
    c = jnp.sum(gt.astype(jnp.int32), axis=1, keepdims=True)
    top = (jnp.sum(jnp.where(gt, v, 0.0), axis=1, keepdims=True)
           + tval * (k - c).astype(jnp.float32))
    top = jnp.where(k >= _A, sumv, jnp.where(k == 0, 0.0, top))

    d = lp_ref[...] - lt_ref[...]
    ad = jnp.abs(d)
    sl1 = jnp.where(ad < 1.0, 0.5 * d * d, ad - 0.5)
    sl1a = jnp.sum(sl1.reshape(4, _N, _A), axis=0)
    loc_loss = jnp.sum(sl1a * posf)

    cls_sum = jnp.sum(pcl + top)
    num_pos = jnp.sum(np_i).astype(jnp.float32)
    out_ref[...] = ((loc_loss + cls_sum) / num_pos).reshape(1, 1)


@functools.partial(jax.jit)
def kernel(loc_preds, loc_targets, cls_preds, cls_targets):
    ct3 = cls_targets.reshape(_N, _A, 1)
    lse3, tl3 = pl.pallas_call(
        _phase1_body,
        grid=(_N,),
        in_specs=[
            pl.BlockSpec((1, _A, _C), lambda n: (n, 0, 0)),
            pl.BlockSpec((1, _A, 1), lambda n: (n, 0, 0)),
        ],
        out_specs=[
            pl.BlockSpec((1, _A, 1), lambda n: (n, 0, 0)),
            pl.BlockSpec((1, _A, 1), lambda n: (n, 0, 0)),
        ],
        out_shape=[
            jax.ShapeDtypeStruct((_N, _A, 1), jnp.float32),
            jax.ShapeDtypeStruct((_N, _A, 1), jnp.float32),
        ],
    )(cls_preds, ct3)

    lp2 = loc_preds.transpose(2, 0, 1).reshape(4 * _N, _A)
    lt2 = loc_targets.transpose(2, 0, 1).reshape(4 * _N, _A)
    out = pl.pallas_call(
        _phase2_body,
        out_shape=jax.ShapeDtypeStruct((1, 1), jnp.float32),
    )(lse3.reshape(_N, _A), tl3.reshape(_N, _A), cls_targets, lp2, lt2)
    return out[0, 0]
